# Initial kernel scaffold; baseline (speedup 1.0000x reference)
import functools

import jax
import jax.numpy as jnp
from jax import lax
from jax.experimental import pallas as pl
from jax.experimental.pallas import tpu as pltpu
from jax.experimental.pallas import tpu_sc as plsc

LATENT = 128
N_POS = 2000
N_NEG = 200
N_EDGE = N_POS + N_NEG
NC, NS = 2, 16
NW = NC * NS
PAD = 2304
B_PER_W = 2 * PAD // NW
N_CHUNKS = 2
CHUNK = B_PER_W // N_CHUNKS

_sc_mesh = plsc.VectorSubcoreMesh(core_axis_name="c", subcore_axis_name="s")


@functools.partial(
    pl.kernel,
    out_type=jax.ShapeDtypeStruct((NW, N_CHUNKS, CHUNK, LATENT), jnp.float32),
    mesh=_sc_mesh,
    scratch_types=[
        pltpu.VMEM((N_CHUNKS, CHUNK), jnp.int32),
        pltpu.VMEM((N_CHUNKS, CHUNK, LATENT), jnp.float32),
        pltpu.SemaphoreType.DMA,
    ],
)
def _sc_gather(table_hbm, idx_hbm, out_hbm, idx_v, rows_v, sem):
    wid = lax.axis_index("s") * NC + lax.axis_index("c")
    pltpu.sync_copy(idx_hbm.at[wid], idx_v)
    copies = [
        pltpu.async_copy(table_hbm.at[idx_v.at[j]], rows_v.at[j], sem)
        for j in range(N_CHUNKS)
    ]
    for c in copies:
        c.wait()
    pltpu.sync_copy(rows_v, out_hbm.at[wid])


def _tc_mlp_loss(g_ref, w1_ref, b1_ref, w2_ref, b2_ref, out_ref):
    src = g_ref[0]
    tgt = g_ref[1]
    w1 = w1_ref[...]
    h = lax.dot_general(src, w1[:, :LATENT], (((1,), (1,)), ((), ())),
                        preferred_element_type=jnp.float32)
    h = h + lax.dot_general(tgt, w1[:, LATENT:], (((1,), (1,)), ((), ())),
                            preferred_element_type=jnp.float32)
    h = jnp.maximum(h + b1_ref[...], 0.0)
    s = lax.dot_general(h, w2_ref[...], (((1,), (1,)), ((), ())),
                        preferred_element_type=jnp.float32)
    s = s + b2_ref[...]
    rows = lax.broadcasted_iota(jnp.int32, (PAD, 1), 0)
    label = (rows < N_POS).astype(jnp.float32)
    per = jnp.maximum(s, 0.0) - s * label + jnp.log1p(jnp.exp(-jnp.abs(s)))
    per = jnp.where(rows < N_EDGE, per, 0.0)
    out_ref[0, 0] = jnp.sum(per) * (1.0 / N_EDGE)


def kernel(v_gene, pos_edge_index, neg_edge_index, W1, b1, W2, b2):
    src = jnp.concatenate([pos_edge_index[0], neg_edge_index[0]]).astype(jnp.int32)
    tgt = jnp.concatenate([pos_edge_index[1], neg_edge_index[1]]).astype(jnp.int32)
    pad = jnp.zeros((PAD - N_EDGE,), jnp.int32)
    idx_all = jnp.concatenate([src, pad, tgt, pad]).reshape(NW, N_CHUNKS, CHUNK)

    gathered = _sc_gather(v_gene, idx_all)
    g = gathered.reshape(2, PAD, LATENT)

    loss = pl.pallas_call(
        _tc_mlp_loss,
        out_shape=jax.ShapeDtypeStruct((1, 1), jnp.float32),
        out_specs=pl.BlockSpec(memory_space=pltpu.SMEM),
    )(g, W1, b1.reshape(1, LATENT), W2, b2.reshape(1, 1))
    return loss[0, 0]


# speedup vs baseline: 2.4020x; 2.4020x over previous
import functools

import jax
import jax.numpy as jnp
from jax import lax
from jax.experimental import pallas as pl
from jax.experimental.pallas import tpu as pltpu
from jax.experimental.pallas import tpu_sc as plsc

LATENT = 128
N_POS = 2000
N_NEG = 200
N_EDGE = N_POS + N_NEG
NC, NS = 2, 16
NW = NC * NS
PAD = 2304
B_PER_W = 2 * PAD // NW
N_CHUNKS = 2
CHUNK = B_PER_W // N_CHUNKS

@functools.cache
def _make_sc_gather():
    mesh = plsc.VectorSubcoreMesh(core_axis_name="c", subcore_axis_name="s")

    @functools.partial(
        pl.kernel,
        out_type=jax.ShapeDtypeStruct((NW, N_CHUNKS, CHUNK, LATENT), jnp.float32),
        mesh=mesh,
        scratch_types=[
            pltpu.VMEM((N_CHUNKS, CHUNK), jnp.int32),
            pltpu.VMEM((N_CHUNKS, CHUNK, LATENT), jnp.float32),
            pltpu.SemaphoreType.DMA,
        ],
    )
    def _sc_gather(table_hbm, idx_hbm, out_hbm, idx_v, rows_v, sem):
        wid = lax.axis_index("s") * NC + lax.axis_index("c")
        pltpu.sync_copy(idx_hbm.at[wid], idx_v)
        copies = [
            pltpu.async_copy(table_hbm.at[idx_v.at[j]], rows_v.at[j], sem)
            for j in range(N_CHUNKS)
        ]
        for c in copies:
            c.wait()
        pltpu.sync_copy(rows_v, out_hbm.at[wid])

    return _sc_gather


def _tc_mlp_loss(g_ref, w1_ref, b1_ref, w2_ref, b2_ref, out_ref):
    src = g_ref[0]
    tgt = g_ref[1]
    w1 = w1_ref[...]
    h = lax.dot_general(src, w1[:, :LATENT], (((1,), (1,)), ((), ())),
                        preferred_element_type=jnp.float32)
    h = h + lax.dot_general(tgt, w1[:, LATENT:], (((1,), (1,)), ((), ())),
                            preferred_element_type=jnp.float32)
    h = jnp.maximum(h + b1_ref[...], 0.0)
    w2b = jnp.broadcast_to(w2_ref[...], (LATENT, LATENT))
    s = lax.dot_general(h, w2b, (((1,), (1,)), ((), ())),
                        preferred_element_type=jnp.float32)
    s = s + b2_ref[0, 0]
    rows = lax.broadcasted_iota(jnp.int32, (PAD, LATENT), 0)
    label = (rows < N_POS).astype(jnp.float32)
    per = jnp.maximum(s, 0.0) - s * label + jnp.log1p(jnp.exp(-jnp.abs(s)))
    per = jnp.where(rows < N_EDGE, per, 0.0)
    out_ref[0, 0] = jnp.sum(per) * (1.0 / (N_EDGE * LATENT))


def kernel(v_gene, pos_edge_index, neg_edge_index, W1, b1, W2, b2):
    src = jnp.concatenate([pos_edge_index[0], neg_edge_index[0]]).astype(jnp.int32)
    tgt = jnp.concatenate([pos_edge_index[1], neg_edge_index[1]]).astype(jnp.int32)
    pad = jnp.zeros((PAD - N_EDGE,), jnp.int32)
    idx_all = jnp.concatenate([src, pad, tgt, pad]).reshape(NW, N_CHUNKS, CHUNK)

    gathered = _make_sc_gather()(v_gene, idx_all)
    g = gathered.reshape(2, PAD, LATENT)

    loss = pl.pallas_call(
        _tc_mlp_loss,
        out_shape=jax.ShapeDtypeStruct((1, 1), jnp.float32),
        in_specs=[
            pl.BlockSpec(memory_space=pltpu.VMEM),
            pl.BlockSpec(memory_space=pltpu.VMEM),
            pl.BlockSpec(memory_space=pltpu.VMEM),
            pl.BlockSpec(memory_space=pltpu.VMEM),
            pl.BlockSpec(memory_space=pltpu.SMEM),
        ],
        out_specs=pl.BlockSpec(memory_space=pltpu.SMEM),
    )(g, W1, b1.reshape(1, LATENT), W2, b2.reshape(1, 1))
    return loss[0, 0]


# speedup vs baseline: 3.1558x; 1.3138x over previous
import functools

import jax
import jax.numpy as jnp
from jax import lax
from jax.experimental import pallas as pl
from jax.experimental.pallas import tpu as pltpu
from jax.experimental.pallas import tpu_sc as plsc

LATENT = 128
N_POS = 2000
N_NEG = 200
N_EDGE = N_POS + N_NEG
NC, NS = 2, 16
NW = NC * NS
POS_Q = 64
NEG_Q = 8
CHUNK = POS_Q + NEG_Q
PAD = NW * CHUNK
POS_LAST = N_POS - POS_Q
NEG_LAST = N_NEG - NEG_Q
POS_DUP = POS_Q * NW - N_POS
NEG_FULL = N_NEG // NEG_Q


@functools.cache
def _make_sc_gather():
    mesh = plsc.VectorSubcoreMesh(core_axis_name="c", subcore_axis_name="s")

    @functools.partial(
        pl.kernel,
        out_type=jax.ShapeDtypeStruct((2, NW, CHUNK, LATENT), jnp.float32),
        mesh=mesh,
        scratch_types=[
            pltpu.VMEM((2, CHUNK), jnp.int32),
            pltpu.VMEM((2, CHUNK, LATENT), jnp.float32),
            pltpu.SemaphoreType.DMA,
            pltpu.SemaphoreType.DMA,
        ],
    )
    def _sc_gather(table_hbm, pos_hbm, neg_hbm, out_hbm, idx_v, rows_v,
                   sem_i, sem_g):
        wid = lax.axis_index("s") * NC + lax.axis_index("c")
        off_p = jnp.minimum(wid * POS_Q, POS_LAST)
        off_n = jnp.minimum(wid * NEG_Q, NEG_LAST)
        loads = []
        for h in range(2):
            loads.append(pltpu.async_copy(
                pos_hbm.at[pl.ds(h * N_POS + off_p, POS_Q)],
                idx_v.at[h, pl.ds(0, POS_Q)], sem_i))
            loads.append(pltpu.async_copy(
                neg_hbm.at[pl.ds(h * N_NEG + off_n, NEG_Q)],
                idx_v.at[h, pl.ds(POS_Q, NEG_Q)], sem_i))
        for c in loads:
            c.wait()
        gathers = [
            pltpu.async_copy(table_hbm.at[idx_v.at[h]], rows_v.at[h], sem_g)
            for h in range(2)
        ]
        for h in range(2):
            gathers[h].wait()
            pltpu.sync_copy(rows_v.at[h], out_hbm.at[h, wid])

    return _sc_gather


def _tc_mlp_loss(g_ref, w1_ref, b1_ref, w2_ref, b2_ref, out_ref):
    src = g_ref[0]
    tgt = g_ref[1]
    w1 = w1_ref[...]
    h = lax.dot_general(src, w1[:, :LATENT], (((1,), (1,)), ((), ())),
                        preferred_element_type=jnp.float32)
    h = h + lax.dot_general(tgt, w1[:, LATENT:], (((1,), (1,)), ((), ())),
                            preferred_element_type=jnp.float32)
    h = jnp.maximum(h + b1_ref[...], 0.0)
    w2b = jnp.broadcast_to(w2_ref[...], (LATENT, LATENT))
    s = lax.dot_general(h, w2b, (((1,), (1,)), ((), ())),
                        preferred_element_type=jnp.float32)
    s = s + b2_ref[0, 0]
    rows = lax.broadcasted_iota(jnp.int32, (PAD, LATENT), 0)
    w = rows // CHUNK
    k = rows - w * CHUNK
    is_pos = k < POS_Q
    valid = jnp.logical_or(
        jnp.logical_and(is_pos, jnp.logical_or(w < NW - 1, k >= POS_DUP)),
        jnp.logical_and(jnp.logical_not(is_pos), w < NEG_FULL),
    )
    label = is_pos.astype(jnp.float32)
    per = jnp.maximum(s, 0.0) - s * label + jnp.log1p(jnp.exp(-jnp.abs(s)))
    per = jnp.where(valid, per, 0.0)
    out_ref[0, 0] = jnp.sum(per) * (1.0 / (N_EDGE * LATENT))


def kernel(v_gene, pos_edge_index, neg_edge_index, W1, b1, W2, b2):
    gathered = _make_sc_gather()(
        v_gene, pos_edge_index.astype(jnp.int32).reshape(-1),
        neg_edge_index.astype(jnp.int32).reshape(-1))
    g = gathered.reshape(2, PAD, LATENT)

    loss = pl.pallas_call(
        _tc_mlp_loss,
        out_shape=jax.ShapeDtypeStruct((1, 1), jnp.float32),
        in_specs=[
            pl.BlockSpec(memory_space=pltpu.VMEM),
            pl.BlockSpec(memory_space=pltpu.VMEM),
            pl.BlockSpec(memory_space=pltpu.VMEM),
            pl.BlockSpec(memory_space=pltpu.VMEM),
            pl.BlockSpec(memory_space=pltpu.SMEM),
        ],
        out_specs=pl.BlockSpec(memory_space=pltpu.SMEM),
    )(g, W1, b1.reshape(1, LATENT), W2, b2.reshape(1, 1))
    return loss[0, 0]


# speedup vs baseline: 3.1634x; 1.0024x over previous
import functools

import jax
import jax.numpy as jnp
from jax import lax
from jax.experimental import pallas as pl
from jax.experimental.pallas import tpu as pltpu
from jax.experimental.pallas import tpu_sc as plsc

LATENT = 128
N_POS = 2000
N_NEG = 200
N_EDGE = N_POS + N_NEG
NC, NS = 2, 16
NW = NC * NS
POS_Q = 64
NEG_Q = 8
CHUNK = POS_Q + NEG_Q
PAD = NW * CHUNK
POS_LAST = N_POS - POS_Q
NEG_LAST = N_NEG - NEG_Q
POS_DUP = POS_Q * NW - N_POS
NEG_FULL = N_NEG // NEG_Q


@functools.cache
def _make_sc_gather():
    mesh = plsc.VectorSubcoreMesh(core_axis_name="c", subcore_axis_name="s")

    @functools.partial(
        pl.kernel,
        out_type=jax.ShapeDtypeStruct((2, NW, CHUNK, LATENT), jnp.float32),
        mesh=mesh,
        scratch_types=[
            pltpu.VMEM((2, CHUNK), jnp.int32),
            pltpu.VMEM((2, CHUNK, LATENT), jnp.float32),
            pltpu.SemaphoreType.DMA,
            pltpu.SemaphoreType.DMA,
        ],
    )
    def _sc_gather(table_hbm, edge_hbm, out_hbm, idx_v, rows_v,
                   sem_i, sem_g):
        wid = lax.axis_index("s") * NC + lax.axis_index("c")
        off_p = jnp.minimum(wid * POS_Q, POS_LAST)
        off_n = jnp.minimum(wid * NEG_Q, NEG_LAST)
        loads = []
        for h in range(2):
            loads.append(pltpu.async_copy(
                edge_hbm.at[pl.ds(h * N_POS + off_p, POS_Q)],
                idx_v.at[h, pl.ds(0, POS_Q)], sem_i))
            loads.append(pltpu.async_copy(
                edge_hbm.at[pl.ds(2 * N_POS + h * N_NEG + off_n, NEG_Q)],
                idx_v.at[h, pl.ds(POS_Q, NEG_Q)], sem_i))
        for c in loads:
            c.wait()
        gathers = [
            pltpu.async_copy(table_hbm.at[idx_v.at[h]], rows_v.at[h], sem_g)
            for h in range(2)
        ]
        for h in range(2):
            gathers[h].wait()
            pltpu.sync_copy(rows_v.at[h], out_hbm.at[h, wid])

    return _sc_gather


def _tc_mlp_loss(g_ref, w1_ref, b1_ref, w2_ref, b2_ref, out_ref):
    src = g_ref[0]
    tgt = g_ref[1]
    w1 = w1_ref[...]
    h = lax.dot_general(src, w1[:, :LATENT], (((1,), (1,)), ((), ())),
                        preferred_element_type=jnp.float32)
    h = h + lax.dot_general(tgt, w1[:, LATENT:], (((1,), (1,)), ((), ())),
                            preferred_element_type=jnp.float32)
    h = jnp.maximum(h + b1_ref[...], 0.0)
    w2b = jnp.broadcast_to(w2_ref[...], (LATENT, LATENT))
    s = lax.dot_general(h, w2b, (((1,), (1,)), ((), ())),
                        preferred_element_type=jnp.float32)
    s = s + b2_ref[0, 0]
    rows = lax.broadcasted_iota(jnp.int32, (PAD, LATENT), 0)
    w = rows // CHUNK
    k = rows - w * CHUNK
    is_pos = k < POS_Q
    valid = jnp.logical_or(
        jnp.logical_and(is_pos, jnp.logical_or(w < NW - 1, k >= POS_DUP)),
        jnp.logical_and(jnp.logical_not(is_pos), w < NEG_FULL),
    )
    label = is_pos.astype(jnp.float32)
    per = jnp.maximum(s, 0.0) - s * label + jnp.log1p(jnp.exp(-jnp.abs(s)))
    per = jnp.where(valid, per, 0.0)
    out_ref[0, 0] = jnp.sum(per) * (1.0 / (N_EDGE * LATENT))


def kernel(v_gene, pos_edge_index, neg_edge_index, W1, b1, W2, b2):
    edges = jnp.concatenate([
        pos_edge_index.astype(jnp.int32).reshape(-1),
        neg_edge_index.astype(jnp.int32).reshape(-1),
    ])
    gathered = _make_sc_gather()(v_gene, edges)
    g = gathered.reshape(2, PAD, LATENT)

    loss = pl.pallas_call(
        _tc_mlp_loss,
        out_shape=jax.ShapeDtypeStruct((1, 1), jnp.float32),
        in_specs=[
            pl.BlockSpec(memory_space=pltpu.VMEM),
            pl.BlockSpec(memory_space=pltpu.VMEM),
            pl.BlockSpec(memory_space=pltpu.VMEM),
            pl.BlockSpec(memory_space=pltpu.VMEM),
            pl.BlockSpec(memory_space=pltpu.SMEM),
        ],
        out_specs=pl.BlockSpec(memory_space=pltpu.SMEM),
    )(g, W1, b1.reshape(1, LATENT), W2, b2.reshape(1, 1))
    return loss[0, 0]
